# baseline (speedup 1.0000x reference)
import functools

import jax
import jax.numpy as jnp
from jax import lax
from jax.experimental import pallas as pl
from jax.experimental.pallas import tpu as pltpu
from jax.experimental.pallas import tpu_sc as plsc

EPS = 1e-5

NC = 2
NS = 16
CHUNK = 80


def _mesh():
    return plsc.VectorSubcoreMesh(core_axis_name="c", subcore_axis_name="s",
                                  num_cores=NC, num_subcores=NS)


def _sc_params():
    return pltpu.CompilerParams(use_tc_tiling_on_sc=False)


def _deg_kernel(n_pad, e, src2_hbm, dst2_hbm, od_out, id_out,
                sidx, didx, ones_v, zbuf, acc_od, acc_id, sem):
    c = lax.axis_index("c")
    s = lax.axis_index("s")
    rows_pt = n_pad // NS
    nchunk = (e // (NC * NS)) // CHUNK
    wid = s * NC + c
    base_k = wid * nchunk

    @pl.loop(0, rows_pt // 16)
    def _z(i):
        zbuf[pl.ds(i * 16, 16)] = jnp.zeros((16,), jnp.float32)

    @pl.loop(0, CHUNK // 16)
    def _o(i):
        ones_v[pl.ds(i * 16, 16)] = jnp.ones((16,), jnp.float32)

    pltpu.sync_copy(src2_hbm.at[pl.ds(base_k, nchunk)], sidx)
    pltpu.sync_copy(dst2_hbm.at[pl.ds(base_k, nchunk)], didx)
    pltpu.sync_copy(zbuf, acc_od.at[pl.ds(s * rows_pt, rows_pt)])
    pltpu.sync_copy(zbuf, acc_id.at[pl.ds(s * rows_pt, rows_pt)])
    plsc.subcore_barrier()

    def _scat(k):
        pltpu.async_copy(ones_v, acc_od.at[sidx.at[k]], sem, add=True)
        pltpu.async_copy(ones_v, acc_id.at[didx.at[k]], sem, add=True)

    def _drain(k):
        pltpu.make_async_copy(ones_v, acc_od.at[sidx.at[k]], sem).wait()
        pltpu.make_async_copy(ones_v, acc_id.at[didx.at[k]], sem).wait()

    _scat(0)

    @pl.loop(1, nchunk)
    def _chunk(k):
        _scat(k)
        _drain(k)

    _drain(0)
    plsc.subcore_barrier()
    pltpu.sync_copy(acc_od.at[pl.ds(s * rows_pt, rows_pt)],
                    od_out.at[c, pl.ds(s * rows_pt, rows_pt)])
    pltpu.sync_copy(acc_id.at[pl.ds(s * rows_pt, rows_pt)],
                    id_out.at[c, pl.ds(s * rows_pt, rows_pt)])


def _make_deg(n_pad, e):
    nchunk = (e // (NC * NS)) // CHUNK
    return pl.kernel(
        functools.partial(_deg_kernel, n_pad, e),
        out_type=(jax.ShapeDtypeStruct((NC, n_pad), jnp.float32),
                  jax.ShapeDtypeStruct((NC, n_pad), jnp.float32)),
        mesh=_mesh(),
        compiler_params=_sc_params(),
        scratch_types=[
            pltpu.VMEM((nchunk, CHUNK), jnp.int32),
            pltpu.VMEM((nchunk, CHUNK), jnp.int32),
            pltpu.VMEM((CHUNK,), jnp.float32),
            pltpu.VMEM((n_pad // NS,), jnp.float32),
            pltpu.VMEM_SHARED((n_pad,), jnp.float32),
            pltpu.VMEM_SHARED((n_pad,), jnp.float32),
            pltpu.SemaphoreType.DMA,
        ],
    )


def _agg_kernel(n_pad, e, h_dim, h_hbm, src2_hbm, dst2_hbm, out,
                sidx, didx, rows, zbuf, acc, sem0, sem1, sem_s):
    c = lax.axis_index("c")
    s = lax.axis_index("s")
    rows_pt = n_pad // NS
    zrows = zbuf.shape[0]
    nchunk = (e // (NC * NS)) // CHUNK
    half = CHUNK // 2
    wid = s * NC + c
    base_k = wid * nchunk

    @pl.loop(0, zrows)
    def _zr(r):
        @pl.loop(0, h_dim // 16)
        def _zc(i):
            zbuf[r, pl.ds(i * 16, 16)] = jnp.zeros((16,), jnp.float32)

    pltpu.sync_copy(src2_hbm.at[pl.ds(base_k, nchunk)], sidx)
    pltpu.sync_copy(dst2_hbm.at[pl.ds(2 * base_k, 2 * nchunk)], didx)

    @pl.loop(0, rows_pt // zrows)
    def _zcopy(j):
        pltpu.sync_copy(zbuf, acc.at[pl.ds(s * rows_pt + j * zrows, zrows)])

    plsc.subcore_barrier()

    sems = (sem0, sem1)

    def _gather(k, b):
        pltpu.async_copy(h_hbm.at[sidx.at[k]], rows.at[b], sems[b])

    def _gwait(k, b):
        pltpu.make_async_copy(h_hbm.at[sidx.at[k]], rows.at[b],
                              sems[b]).wait()

    def _scat(k, b):
        pltpu.async_copy(rows.at[b, pl.ds(0, half)],
                         acc.at[didx.at[2 * k]], sem_s, add=True)
        pltpu.async_copy(rows.at[b, pl.ds(half, half)],
                         acc.at[didx.at[2 * k + 1]], sem_s, add=True)
        pltpu.make_async_copy(rows.at[b, pl.ds(0, half)],
                              acc.at[didx.at[2 * k]], sem_s).wait()
        pltpu.make_async_copy(rows.at[b, pl.ds(half, half)],
                              acc.at[didx.at[2 * k + 1]], sem_s).wait()

    _gather(0, 0)

    @pl.loop(0, (nchunk - 1) // 2)
    def _pipe(g):
        k = 2 * g
        _gather(k + 1, 1)
        _gwait(k, 0)
        _scat(k, 0)
        _gather(k + 2, 0)
        _gwait(k + 1, 1)
        _scat(k + 1, 1)

    _gwait(nchunk - 1, 0)
    _scat(nchunk - 1, 0)

    plsc.subcore_barrier()
    pltpu.sync_copy(acc.at[pl.ds(s * rows_pt, rows_pt)],
                    out.at[c, pl.ds(s * rows_pt, rows_pt)])


def _make_agg(n_pad, e, h_dim):
    zrows = 8
    nchunk = (e // (NC * NS)) // CHUNK
    return pl.kernel(
        functools.partial(_agg_kernel, n_pad, e, h_dim),
        out_type=jax.ShapeDtypeStruct((NC, n_pad, h_dim), jnp.float32),
        mesh=_mesh(),
        compiler_params=_sc_params(),
        scratch_types=[
            pltpu.VMEM((nchunk, CHUNK), jnp.int32),
            pltpu.VMEM((2 * nchunk, CHUNK // 2), jnp.int32),
            pltpu.VMEM((2, CHUNK, h_dim), jnp.float32),
            pltpu.VMEM((zrows, h_dim), jnp.float32),
            pltpu.VMEM_SHARED((n_pad, h_dim), jnp.float32),
            pltpu.SemaphoreType.DMA,
            pltpu.SemaphoreType.DMA,
            pltpu.SemaphoreType.DMA,
        ],
    )


def _tc_call(body, out_shape, n_in):
    return pl.pallas_call(
        body,
        out_shape=out_shape,
        in_specs=[pl.BlockSpec(memory_space=pltpu.VMEM)] * n_in,
        out_specs=pl.BlockSpec(memory_space=pltpu.VMEM),
    )


def _mm_kernel(x_ref, w_ref, out_ref):
    out_ref[...] = jnp.dot(x_ref[...], w_ref[...],
                           preferred_element_type=jnp.float32)


def _scale_kernel(n, mm_ref, dod_ref, out_ref):
    dout = dod_ref[0, :n] + dod_ref[1, :n]
    r = lax.rsqrt(jnp.maximum(dout, 1.0))
    out_ref[...] = mm_ref[...] * r[:, None]


def _mid_kernel(n, p_ref, did_ref, dod_ref, b_ref, g_ref, be_ref, w_ref,
                out_ref):
    p = p_ref[0, :n, :] + p_ref[1, :n, :]
    din = did_ref[0, :n] + did_ref[1, :n]
    y = p * lax.rsqrt(jnp.maximum(din, 1.0))[:, None] + b_ref[...]
    mean = jnp.mean(y, axis=0, keepdims=True)
    var = jnp.mean((y - mean) ** 2, axis=0, keepdims=True)
    z = g_ref[...] * (y - mean) / jnp.sqrt(var + EPS) + be_ref[...]
    z = jnp.maximum(z, 0.0)
    dout = dod_ref[0, :n] + dod_ref[1, :n]
    z = z * lax.rsqrt(jnp.maximum(dout, 1.0))[:, None]
    out_ref[...] = jnp.dot(z, w_ref[...], preferred_element_type=jnp.float32)


def _last_kernel(n, p_ref, did_ref, b_ref, out_ref):
    p = p_ref[0, :n, :] + p_ref[1, :n, :]
    din = did_ref[0, :n] + did_ref[1, :n]
    out_ref[...] = (p * lax.rsqrt(jnp.maximum(din, 1.0))[:, None]
                    + b_ref[...])


def kernel(edge_index, input_features, W0, b0, g0, be0, W1, b1, g1, be1,
           W2, b2):
    n, d_in = input_features.shape
    e = edge_index.shape[1]
    h = W0.shape[1]
    d_out = W2.shape[1]
    n_pad = ((n + 8 * NS - 1) // (8 * NS)) * (8 * NS)

    src2 = edge_index[0].reshape(e // CHUNK, CHUNK)
    dst2 = edge_index[1].reshape(e // CHUNK, CHUNK)
    dst4 = edge_index[1].reshape(2 * (e // CHUNK), CHUNK // 2)

    deg = _make_deg(n_pad, e)
    agg_h = _make_agg(n_pad, e, h)
    agg_o = _make_agg(n_pad, e, d_out)

    mm0 = _tc_call(_mm_kernel,
                   jax.ShapeDtypeStruct((n, h), jnp.float32), 2)(
                       input_features, W0)
    od_p, id_p = deg(src2, dst2)

    b0r, g0r, be0r = b0.reshape(1, -1), g0.reshape(1, -1), be0.reshape(1, -1)
    b1r, g1r, be1r = b1.reshape(1, -1), g1.reshape(1, -1), be1.reshape(1, -1)
    b2r = b2.reshape(1, -1)

    h0 = _tc_call(functools.partial(_scale_kernel, n),
                  jax.ShapeDtypeStruct((n, h), jnp.float32), 2)(
                      mm0, od_p)
    p0 = agg_h(h0, src2, dst4)
    h1 = _tc_call(functools.partial(_mid_kernel, n),
                  jax.ShapeDtypeStruct((n, h), jnp.float32), 7)(
                      p0, id_p, od_p, b0r, g0r, be0r, W1)
    p1 = agg_h(h1, src2, dst4)
    h2 = _tc_call(functools.partial(_mid_kernel, n),
                  jax.ShapeDtypeStruct((n, d_out), jnp.float32), 7)(
                      p1, id_p, od_p, b1r, g1r, be1r, W2)
    p2 = agg_o(h2, src2, dst4)
    out = _tc_call(functools.partial(_last_kernel, n),
                   jax.ShapeDtypeStruct((n, d_out), jnp.float32), 3)(
                       p2, id_p, b2r)
    return out


# speedup vs baseline: 1.0009x; 1.0009x over previous
import functools

import jax
import jax.numpy as jnp
from jax import lax
from jax.experimental import pallas as pl
from jax.experimental.pallas import tpu as pltpu
from jax.experimental.pallas import tpu_sc as plsc

EPS = 1e-5

NC = 2
NS = 16
CHUNK = 80


def _mesh():
    return plsc.VectorSubcoreMesh(core_axis_name="c", subcore_axis_name="s",
                                  num_cores=NC, num_subcores=NS)


def _sc_params():
    return pltpu.CompilerParams(use_tc_tiling_on_sc=False)


def _deg_kernel(n_pad, e, src2_hbm, dst2_hbm, od_out, id_out,
                sidx, didx, ones_v, zbuf, acc_od, acc_id, sem):
    c = lax.axis_index("c")
    s = lax.axis_index("s")
    rows_pt = n_pad // NS
    nchunk = (e // (NC * NS)) // CHUNK
    wid = s * NC + c
    base_k = wid * nchunk

    @pl.loop(0, rows_pt // 16)
    def _z(i):
        zbuf[pl.ds(i * 16, 16)] = jnp.zeros((16,), jnp.float32)

    @pl.loop(0, CHUNK // 16)
    def _o(i):
        ones_v[pl.ds(i * 16, 16)] = jnp.ones((16,), jnp.float32)

    pltpu.sync_copy(src2_hbm.at[pl.ds(base_k, nchunk)], sidx)
    pltpu.sync_copy(dst2_hbm.at[pl.ds(base_k, nchunk)], didx)
    pltpu.sync_copy(zbuf, acc_od.at[pl.ds(s * rows_pt, rows_pt)])
    pltpu.sync_copy(zbuf, acc_id.at[pl.ds(s * rows_pt, rows_pt)])
    plsc.subcore_barrier()

    def _scat(k):
        pltpu.async_copy(ones_v, acc_od.at[sidx.at[k]], sem, add=True)
        pltpu.async_copy(ones_v, acc_id.at[didx.at[k]], sem, add=True)

    def _drain(k):
        pltpu.make_async_copy(ones_v, acc_od.at[sidx.at[k]], sem).wait()
        pltpu.make_async_copy(ones_v, acc_id.at[didx.at[k]], sem).wait()

    _scat(0)

    @pl.loop(1, nchunk)
    def _chunk(k):
        _scat(k)
        _drain(k)

    _drain(0)
    plsc.subcore_barrier()
    pltpu.sync_copy(acc_od.at[pl.ds(s * rows_pt, rows_pt)],
                    od_out.at[c, pl.ds(s * rows_pt, rows_pt)])
    pltpu.sync_copy(acc_id.at[pl.ds(s * rows_pt, rows_pt)],
                    id_out.at[c, pl.ds(s * rows_pt, rows_pt)])


def _make_deg(n_pad, e):
    nchunk = (e // (NC * NS)) // CHUNK
    return pl.kernel(
        functools.partial(_deg_kernel, n_pad, e),
        out_type=(jax.ShapeDtypeStruct((NC, n_pad), jnp.float32),
                  jax.ShapeDtypeStruct((NC, n_pad), jnp.float32)),
        mesh=_mesh(),
        compiler_params=_sc_params(),
        scratch_types=[
            pltpu.VMEM((nchunk, CHUNK), jnp.int32),
            pltpu.VMEM((nchunk, CHUNK), jnp.int32),
            pltpu.VMEM((CHUNK,), jnp.float32),
            pltpu.VMEM((n_pad // NS,), jnp.float32),
            pltpu.VMEM_SHARED((n_pad,), jnp.float32),
            pltpu.VMEM_SHARED((n_pad,), jnp.float32),
            pltpu.SemaphoreType.DMA,
        ],
    )


def _agg_kernel(n_pad, e, h_dim, h_hbm, src2_hbm, dst2_hbm, out,
                sidx, didx, rows, zbuf, acc, sem0, sem1, sem_s):
    c = lax.axis_index("c")
    s = lax.axis_index("s")
    rows_pt = n_pad // NS
    zrows = zbuf.shape[0]
    nchunk = (e // (NC * NS)) // CHUNK
    half = CHUNK // 2
    wid = s * NC + c
    base_k = wid * nchunk

    @pl.loop(0, zrows)
    def _zr(r):
        @pl.loop(0, h_dim // 16)
        def _zc(i):
            zbuf[r, pl.ds(i * 16, 16)] = jnp.zeros((16,), jnp.float32)

    pltpu.sync_copy(src2_hbm.at[pl.ds(2 * base_k, 2 * nchunk)], sidx)
    pltpu.sync_copy(dst2_hbm.at[pl.ds(base_k, nchunk)], didx)

    @pl.loop(0, rows_pt // zrows)
    def _zcopy(j):
        pltpu.sync_copy(zbuf, acc.at[pl.ds(s * rows_pt + j * zrows, zrows)])

    plsc.subcore_barrier()

    sems = (sem0, sem1)

    def _gather(k, b):
        pltpu.async_copy(h_hbm.at[sidx.at[2 * k]],
                         rows.at[b, pl.ds(0, half)], sems[b])
        pltpu.async_copy(h_hbm.at[sidx.at[2 * k + 1]],
                         rows.at[b, pl.ds(half, half)], sems[b])

    def _gwait(k, b):
        pltpu.make_async_copy(h_hbm.at[sidx.at[2 * k]],
                              rows.at[b, pl.ds(0, half)], sems[b]).wait()
        pltpu.make_async_copy(h_hbm.at[sidx.at[2 * k + 1]],
                              rows.at[b, pl.ds(half, half)], sems[b]).wait()

    def _scat(k, b):
        pltpu.sync_copy(rows.at[b], acc.at[didx.at[k]], add=True)

    _gather(0, 0)

    @pl.loop(0, (nchunk - 1) // 2)
    def _pipe(g):
        k = 2 * g
        _gather(k + 1, 1)
        _gwait(k, 0)
        _scat(k, 0)
        _gather(k + 2, 0)
        _gwait(k + 1, 1)
        _scat(k + 1, 1)

    _gwait(nchunk - 1, 0)
    _scat(nchunk - 1, 0)

    plsc.subcore_barrier()
    pltpu.sync_copy(acc.at[pl.ds(s * rows_pt, rows_pt)],
                    out.at[c, pl.ds(s * rows_pt, rows_pt)])


def _make_agg(n_pad, e, h_dim):
    zrows = 8
    nchunk = (e // (NC * NS)) // CHUNK
    return pl.kernel(
        functools.partial(_agg_kernel, n_pad, e, h_dim),
        out_type=jax.ShapeDtypeStruct((NC, n_pad, h_dim), jnp.float32),
        mesh=_mesh(),
        compiler_params=_sc_params(),
        scratch_types=[
            pltpu.VMEM((2 * nchunk, CHUNK // 2), jnp.int32),
            pltpu.VMEM((nchunk, CHUNK), jnp.int32),
            pltpu.VMEM((2, CHUNK, h_dim), jnp.float32),
            pltpu.VMEM((zrows, h_dim), jnp.float32),
            pltpu.VMEM_SHARED((n_pad, h_dim), jnp.float32),
            pltpu.SemaphoreType.DMA,
            pltpu.SemaphoreType.DMA,
            pltpu.SemaphoreType.DMA,
        ],
    )


def _tc_call(body, out_shape, n_in):
    return pl.pallas_call(
        body,
        out_shape=out_shape,
        in_specs=[pl.BlockSpec(memory_space=pltpu.VMEM)] * n_in,
        out_specs=pl.BlockSpec(memory_space=pltpu.VMEM),
    )


def _mm_kernel(x_ref, w_ref, out_ref):
    out_ref[...] = jnp.dot(x_ref[...], w_ref[...],
                           preferred_element_type=jnp.float32)


def _scale_kernel(n, mm_ref, dod_ref, out_ref):
    dout = dod_ref[0, :n] + dod_ref[1, :n]
    r = lax.rsqrt(jnp.maximum(dout, 1.0))
    out_ref[...] = mm_ref[...] * r[:, None]


def _mid_kernel(n, p_ref, did_ref, dod_ref, b_ref, g_ref, be_ref, w_ref,
                out_ref):
    p = p_ref[0, :n, :] + p_ref[1, :n, :]
    din = did_ref[0, :n] + did_ref[1, :n]
    y = p * lax.rsqrt(jnp.maximum(din, 1.0))[:, None] + b_ref[...]
    mean = jnp.mean(y, axis=0, keepdims=True)
    var = jnp.mean((y - mean) ** 2, axis=0, keepdims=True)
    z = g_ref[...] * (y - mean) / jnp.sqrt(var + EPS) + be_ref[...]
    z = jnp.maximum(z, 0.0)
    dout = dod_ref[0, :n] + dod_ref[1, :n]
    z = z * lax.rsqrt(jnp.maximum(dout, 1.0))[:, None]
    out_ref[...] = jnp.dot(z, w_ref[...], preferred_element_type=jnp.float32)


def _last_kernel(n, p_ref, did_ref, b_ref, out_ref):
    p = p_ref[0, :n, :] + p_ref[1, :n, :]
    din = did_ref[0, :n] + did_ref[1, :n]
    out_ref[...] = (p * lax.rsqrt(jnp.maximum(din, 1.0))[:, None]
                    + b_ref[...])


def kernel(edge_index, input_features, W0, b0, g0, be0, W1, b1, g1, be1,
           W2, b2):
    n, d_in = input_features.shape
    e = edge_index.shape[1]
    h = W0.shape[1]
    d_out = W2.shape[1]
    n_pad = ((n + 8 * NS - 1) // (8 * NS)) * (8 * NS)

    src2 = edge_index[0].reshape(e // CHUNK, CHUNK)
    dst2 = edge_index[1].reshape(e // CHUNK, CHUNK)
    src4 = edge_index[0].reshape(2 * (e // CHUNK), CHUNK // 2)

    deg = _make_deg(n_pad, e)
    agg_h = _make_agg(n_pad, e, h)
    agg_o = _make_agg(n_pad, e, d_out)

    mm0 = _tc_call(_mm_kernel,
                   jax.ShapeDtypeStruct((n, h), jnp.float32), 2)(
                       input_features, W0)
    od_p, id_p = deg(src2, dst2)

    b0r, g0r, be0r = b0.reshape(1, -1), g0.reshape(1, -1), be0.reshape(1, -1)
    b1r, g1r, be1r = b1.reshape(1, -1), g1.reshape(1, -1), be1.reshape(1, -1)
    b2r = b2.reshape(1, -1)

    h0 = _tc_call(functools.partial(_scale_kernel, n),
                  jax.ShapeDtypeStruct((n, h), jnp.float32), 2)(
                      mm0, od_p)
    p0 = agg_h(h0, src4, dst2)
    h1 = _tc_call(functools.partial(_mid_kernel, n),
                  jax.ShapeDtypeStruct((n, h), jnp.float32), 7)(
                      p0, id_p, od_p, b0r, g0r, be0r, W1)
    p1 = agg_h(h1, src4, dst2)
    h2 = _tc_call(functools.partial(_mid_kernel, n),
                  jax.ShapeDtypeStruct((n, d_out), jnp.float32), 7)(
                      p1, id_p, od_p, b1r, g1r, be1r, W2)
    p2 = agg_o(h2, src4, dst2)
    out = _tc_call(functools.partial(_last_kernel, n),
                   jax.ShapeDtypeStruct((n, d_out), jnp.float32), 3)(
                       p2, id_p, b2r)
    return out


# speedup vs baseline: 1.0521x; 1.0511x over previous
import functools

import jax
import jax.numpy as jnp
from jax import lax
from jax.experimental import pallas as pl
from jax.experimental.pallas import tpu as pltpu
from jax.experimental.pallas import tpu_sc as plsc

EPS = 1e-5

NC = 2
NS = 16
CHUNK = 80


def _mesh():
    return plsc.VectorSubcoreMesh(core_axis_name="c", subcore_axis_name="s",
                                  num_cores=NC, num_subcores=NS)


def _sc_params():
    return pltpu.CompilerParams(use_tc_tiling_on_sc=False)


def _deg_kernel(n_pad, e, src2_hbm, dst2_hbm, od_out, id_out,
                sidx, didx, ones_v, zbuf, acc_od, acc_id, sem):
    c = lax.axis_index("c")
    s = lax.axis_index("s")
    rows_pt = n_pad // NS
    nchunk = (e // (NC * NS)) // CHUNK
    wid = s * NC + c
    base_k = wid * nchunk

    @pl.loop(0, rows_pt // 16)
    def _z(i):
        zbuf[pl.ds(i * 16, 16)] = jnp.zeros((16,), jnp.float32)

    @pl.loop(0, CHUNK // 16)
    def _o(i):
        ones_v[pl.ds(i * 16, 16)] = jnp.ones((16,), jnp.float32)

    pltpu.sync_copy(src2_hbm.at[pl.ds(base_k, nchunk)], sidx)
    pltpu.sync_copy(dst2_hbm.at[pl.ds(base_k, nchunk)], didx)
    pltpu.sync_copy(zbuf, acc_od.at[pl.ds(s * rows_pt, rows_pt)])
    pltpu.sync_copy(zbuf, acc_id.at[pl.ds(s * rows_pt, rows_pt)])
    plsc.subcore_barrier()

    def _scat(k):
        pltpu.async_copy(ones_v, acc_od.at[sidx.at[k]], sem, add=True)
        pltpu.async_copy(ones_v, acc_id.at[didx.at[k]], sem, add=True)

    def _drain(k):
        pltpu.make_async_copy(ones_v, acc_od.at[sidx.at[k]], sem).wait()
        pltpu.make_async_copy(ones_v, acc_id.at[didx.at[k]], sem).wait()

    _scat(0)

    @pl.loop(1, nchunk)
    def _chunk(k):
        _scat(k)
        _drain(k)

    _drain(0)
    plsc.subcore_barrier()
    pltpu.sync_copy(acc_od.at[pl.ds(s * rows_pt, rows_pt)],
                    od_out.at[c, pl.ds(s * rows_pt, rows_pt)])
    pltpu.sync_copy(acc_id.at[pl.ds(s * rows_pt, rows_pt)],
                    id_out.at[c, pl.ds(s * rows_pt, rows_pt)])


def _make_deg(n_pad, e):
    nchunk = (e // (NC * NS)) // CHUNK
    return pl.kernel(
        functools.partial(_deg_kernel, n_pad, e),
        out_type=(jax.ShapeDtypeStruct((NC, n_pad), jnp.float32),
                  jax.ShapeDtypeStruct((NC, n_pad), jnp.float32)),
        mesh=_mesh(),
        compiler_params=_sc_params(),
        scratch_types=[
            pltpu.VMEM((nchunk, CHUNK), jnp.int32),
            pltpu.VMEM((nchunk, CHUNK), jnp.int32),
            pltpu.VMEM((CHUNK,), jnp.float32),
            pltpu.VMEM((n_pad // NS,), jnp.float32),
            pltpu.VMEM_SHARED((n_pad,), jnp.float32),
            pltpu.VMEM_SHARED((n_pad,), jnp.float32),
            pltpu.SemaphoreType.DMA,
        ],
    )


def _agg_kernel(n_pad, e, h_dim, h_hbm, src3_hbm, dst3_hbm, out,
                sidx, didx, rows, zbuf, acc, sem0, sem1):
    c = lax.axis_index("c")
    s = lax.axis_index("s")
    rows_pt = n_pad // NS
    zrows = zbuf.shape[0]
    nchunk = (e // (NC * NS)) // CHUNK
    wid = s * NC + c

    @pl.loop(0, zrows)
    def _zr(r):
        @pl.loop(0, h_dim // 16)
        def _zc(i):
            zbuf[r, pl.ds(i * 16, 16)] = jnp.zeros((16,), jnp.float32)

    pltpu.sync_copy(src3_hbm.at[wid], sidx)
    pltpu.sync_copy(dst3_hbm.at[wid], didx)

    nfull = rows_pt // zrows
    tail = rows_pt - nfull * zrows

    @pl.loop(0, nfull)
    def _zcopy(j):
        pltpu.sync_copy(zbuf, acc.at[pl.ds(s * rows_pt + j * zrows, zrows)])

    if tail:
        pltpu.sync_copy(zbuf.at[pl.ds(0, tail)],
                        acc.at[pl.ds(s * rows_pt + nfull * zrows, tail)])

    plsc.subcore_barrier()

    sems = (sem0, sem1)

    def _gather(k, b):
        pltpu.async_copy(h_hbm.at[sidx.at[k]], rows.at[b], sems[b])

    def _gwait(k, b):
        pltpu.make_async_copy(h_hbm.at[sidx.at[k]], rows.at[b],
                              sems[b]).wait()

    def _scat(k, b):
        pltpu.sync_copy(rows.at[b], acc.at[didx.at[k]], add=True)

    _gather(0, 0)

    @pl.loop(0, (nchunk - 1) // 2)
    def _pipe(g):
        k = 2 * g
        _gather(k + 1, 1)
        _gwait(k, 0)
        _scat(k, 0)
        _gather(k + 2, 0)
        _gwait(k + 1, 1)
        _scat(k + 1, 1)

    _gwait(nchunk - 1, 0)
    _scat(nchunk - 1, 0)

    plsc.subcore_barrier()
    pltpu.sync_copy(acc.at[pl.ds(s * rows_pt, rows_pt)],
                    out.at[c, pl.ds(s * rows_pt, rows_pt)])


def _make_agg(n_pad, e, h_dim):
    zrows = 64
    nchunk = (e // (NC * NS)) // CHUNK
    return pl.kernel(
        functools.partial(_agg_kernel, n_pad, e, h_dim),
        out_type=jax.ShapeDtypeStruct((NC, n_pad, h_dim), jnp.float32),
        mesh=_mesh(),
        compiler_params=_sc_params(),
        scratch_types=[
            pltpu.VMEM((nchunk, CHUNK), jnp.int32),
            pltpu.VMEM((nchunk, CHUNK), jnp.int32),
            pltpu.VMEM((2, CHUNK, h_dim), jnp.float32),
            pltpu.VMEM((zrows, h_dim), jnp.float32),
            pltpu.VMEM_SHARED((n_pad, h_dim), jnp.float32),
            pltpu.SemaphoreType.DMA,
            pltpu.SemaphoreType.DMA,
        ],
    )


def _tc_call(body, out_shape, n_in):
    return pl.pallas_call(
        body,
        out_shape=out_shape,
        in_specs=[pl.BlockSpec(memory_space=pltpu.VMEM)] * n_in,
        out_specs=pl.BlockSpec(memory_space=pltpu.VMEM),
    )


def _first_kernel(n, x_ref, w_ref, dod_ref, out_ref):
    dout = dod_ref[0, :n] + dod_ref[1, :n]
    r = lax.rsqrt(jnp.maximum(dout, 1.0))
    out_ref[...] = jnp.dot(x_ref[...] * r[:, None], w_ref[...],
                           preferred_element_type=jnp.float32)


def _mid_kernel(n, p_ref, did_ref, dod_ref, b_ref, g_ref, be_ref, w_ref,
                out_ref):
    p = p_ref[0, :n, :] + p_ref[1, :n, :]
    din = did_ref[0, :n] + did_ref[1, :n]
    y = p * lax.rsqrt(jnp.maximum(din, 1.0))[:, None] + b_ref[...]
    mean = jnp.mean(y, axis=0, keepdims=True)
    var = jnp.mean((y - mean) ** 2, axis=0, keepdims=True)
    z = g_ref[...] * (y - mean) / jnp.sqrt(var + EPS) + be_ref[...]
    z = jnp.maximum(z, 0.0)
    dout = dod_ref[0, :n] + dod_ref[1, :n]
    z = z * lax.rsqrt(jnp.maximum(dout, 1.0))[:, None]
    out_ref[...] = jnp.dot(z, w_ref[...], preferred_element_type=jnp.float32)


def _last_kernel(n, d_out, p_ref, r2_ref, b_ref, out_ref):
    half_n = n // 2
    p = p_ref[0, :half_n, :] + p_ref[1, :half_n, :]
    b2 = jnp.concatenate([b_ref[...], b_ref[...]], axis=1)
    out_ref[...] = p * r2_ref[...] + b2


def kernel(edge_index, input_features, W0, b0, g0, be0, W1, b1, g1, be1,
           W2, b2):
    n, d_in = input_features.shape
    e = edge_index.shape[1]
    h = W0.shape[1]
    d_out = W2.shape[1]
    n_pad = ((n + 8 * NS - 1) // (8 * NS)) * (8 * NS)

    nw = NC * NS
    nchunk = (e // nw) // CHUNK
    src2 = edge_index[0].reshape(e // CHUNK, CHUNK)
    dst2 = edge_index[1].reshape(e // CHUNK, CHUNK)
    src3 = edge_index[0].reshape(nw, nchunk, CHUNK)
    dst3 = edge_index[1].reshape(nw, nchunk, CHUNK)

    deg = _make_deg(n_pad, e)
    agg_h = _make_agg(n_pad, e, h)
    agg_o = _make_agg(n_pad, e, d_out)

    od_p, id_p = deg(src2, dst2)

    b0r, g0r, be0r = b0.reshape(1, -1), g0.reshape(1, -1), be0.reshape(1, -1)
    b1r, g1r, be1r = b1.reshape(1, -1), g1.reshape(1, -1), be1.reshape(1, -1)
    b2r = b2.reshape(1, -1)

    h0 = _tc_call(functools.partial(_first_kernel, n),
                  jax.ShapeDtypeStruct((n, h), jnp.float32), 3)(
                      input_features, W0, od_p)
    p0 = agg_h(h0, src3, dst3)
    h1 = _tc_call(functools.partial(_mid_kernel, n),
                  jax.ShapeDtypeStruct((n, h), jnp.float32), 7)(
                      p0, id_p, od_p, b0r, g0r, be0r, W1)
    p1 = agg_h(h1, src3, dst3)
    h2 = _tc_call(functools.partial(_mid_kernel, n),
                  jax.ShapeDtypeStruct((n, d_out), jnp.float32), 7)(
                      p1, id_p, od_p, b1r, g1r, be1r, W2)
    p2 = agg_o(h2, src3, dst3)
    p2v = p2.reshape(NC, n_pad // 2, 2 * d_out)
    rin = lax.rsqrt(jnp.maximum(id_p[0, :n] + id_p[1, :n], 1.0))
    r2v = jnp.broadcast_to(rin[:, None], (n, d_out)).reshape(
        n // 2, 2 * d_out)
    outv = _tc_call(functools.partial(_last_kernel, n, d_out),
                    jax.ShapeDtypeStruct((n // 2, 2 * d_out), jnp.float32),
                    3)(p2v, r2v, b2r)
    return outv.reshape(n, d_out)
